# baseline (speedup 1.0000x reference)
import functools

import jax
import jax.numpy as jnp
from jax import lax
from jax.experimental import pallas as pl
from jax.experimental.pallas import tpu as pltpu
from jax.experimental.pallas import tpu_sc as plsc

EPS = 1e-12


def _make_sc_gather(V, D, B):
    info = plsc.get_sparse_core_info()
    NC, NS = info.num_cores, info.num_subcores
    NW = NC * NS
    assert B % NW == 0
    b_per_w = B // NW
    C = min(b_per_w, 64)
    assert b_per_w % C == 0
    n_chunks = b_per_w // C
    mesh = plsc.VectorSubcoreMesh(core_axis_name="c", subcore_axis_name="s")

    @functools.partial(
        pl.kernel,
        mesh=mesh,
        out_type=jax.ShapeDtypeStruct((B, D), jnp.float32),
        scratch_types=[
            pltpu.VMEM((C,), jnp.int32),
            pltpu.VMEM((C, D), jnp.float32),
            pltpu.SemaphoreType.DMA,
        ],
    )
    def sc_gather(table_hbm, idx_hbm, out_hbm, idx_v, rows_v, sem):
        wid = lax.axis_index("s") * NC + lax.axis_index("c")
        base = wid * b_per_w
        for c in range(n_chunks):
            lo = base + c * C
            pltpu.sync_copy(idx_hbm.at[pl.ds(lo, C)], idx_v)
            pltpu.async_copy(table_hbm.at[idx_v], rows_v, sem).wait()
            pltpu.sync_copy(rows_v, out_hbm.at[pl.ds(lo, C)])

    return sc_gather


def _tc_slice_body(g_ref, p_ref, gamma_ref, beta_ref, o_ref):
    x = g_ref[...] + p_ref[...]
    mean = jnp.mean(x, axis=-1, keepdims=True)
    xc = x - mean
    var = jnp.mean(xc * xc, axis=-1, keepdims=True)
    xhat = xc * lax.rsqrt(var + EPS)
    o_ref[...] = (xhat * gamma_ref[...] + beta_ref[...])[None, :, :]


def _tc_add_ln_slice(buf, b, Bt, gathered, pos_emb, gamma, beta):
    S, D = gathered.shape
    R = 512
    pos_blocks = S // R
    data_specs = [
        pl.BlockSpec((R, D), lambda j: (j, 0)),
        pl.BlockSpec((R, D), lambda j: (j, 0)),
        pl.BlockSpec((1, D), lambda j: (0, 0)),
        pl.BlockSpec((1, D), lambda j: (0, 0)),
    ]
    first = buf is None
    in_specs = data_specs if first else [pl.BlockSpec(memory_space=pl.ANY)] + data_specs
    body = _tc_slice_body if first else (lambda d, *a: _tc_slice_body(*a))
    args = () if first else (buf,)
    return pl.pallas_call(
        body,
        grid=(pos_blocks,),
        in_specs=in_specs,
        out_specs=pl.BlockSpec((1, R, D), lambda j: (b, j, 0)),
        out_shape=jax.ShapeDtypeStruct((Bt, S, D), jnp.float32),
        input_output_aliases={} if first else {0: 0},
    )(*args, gathered, pos_emb, gamma.reshape(1, D), beta.reshape(1, D))


def kernel(input_ids, word_emb, pos_emb, ln_gamma, ln_beta):
    Bt, S = input_ids.shape
    V, D = word_emb.shape
    ids = input_ids.astype(jnp.int32)
    sc_gather = _make_sc_gather(V, D, S)
    gathered = [sc_gather(word_emb, ids[b]) for b in range(Bt)]
    buf = None
    for b in range(Bt):
        buf = _tc_add_ln_slice(buf, b, Bt, gathered[b], pos_emb, ln_gamma, ln_beta)
    return buf


# speedup vs baseline: 1.0846x; 1.0846x over previous
import functools

import jax
import jax.numpy as jnp
from jax import lax
from jax.experimental import pallas as pl
from jax.experimental.pallas import tpu as pltpu
from jax.experimental.pallas import tpu_sc as plsc

EPS = 1e-12


def _make_sc_gather(V, D, B):
    info = plsc.get_sparse_core_info()
    NC, NS = info.num_cores, info.num_subcores
    NW = NC * NS
    assert B % NW == 0
    b_per_w = B // NW
    C = min(b_per_w, 64)
    assert b_per_w % C == 0
    n_chunks = b_per_w // C
    mesh = plsc.VectorSubcoreMesh(core_axis_name="c", subcore_axis_name="s")

    @functools.partial(
        pl.kernel,
        mesh=mesh,
        out_type=jax.ShapeDtypeStruct((B, D), jnp.float32),
        scratch_types=[
            pltpu.VMEM((C,), jnp.int32),
            pltpu.VMEM((C, D), jnp.float32),
            pltpu.SemaphoreType.DMA,
        ],
    )
    def sc_gather(table_hbm, idx_hbm, out_hbm, idx_v, rows_v, sem):
        wid = lax.axis_index("s") * NC + lax.axis_index("c")
        base = wid * b_per_w
        for c in range(n_chunks):
            lo = base + c * C
            pltpu.sync_copy(idx_hbm.at[pl.ds(lo, C)], idx_v)
            pltpu.async_copy(table_hbm.at[idx_v], rows_v, sem).wait()
            pltpu.sync_copy(rows_v, out_hbm.at[pl.ds(lo, C)])

    return sc_gather


def _tc_add_ln_body(g_ref, p_ref, gamma_ref, beta_ref, o_ref):
    x = g_ref[...] + p_ref[...][None, :, :]
    mean = jnp.mean(x, axis=-1, keepdims=True)
    xc = x - mean
    var = jnp.mean(xc * xc, axis=-1, keepdims=True)
    xhat = xc * lax.rsqrt(var + EPS)
    o_ref[...] = xhat * gamma_ref[...] + beta_ref[...]


def _tc_add_ln(gathered3, pos_emb, gamma, beta, R=1024):
    Bt, S, D = gathered3.shape
    pos_blocks = S // R
    return pl.pallas_call(
        _tc_add_ln_body,
        grid=(pos_blocks, Bt),
        in_specs=[
            pl.BlockSpec((1, R, D), lambda j, b: (b, j, 0)),
            pl.BlockSpec((R, D), lambda j, b: (j, 0)),
            pl.BlockSpec((1, D), lambda j, b: (0, 0)),
            pl.BlockSpec((1, D), lambda j, b: (0, 0)),
        ],
        out_specs=pl.BlockSpec((1, R, D), lambda j, b: (b, j, 0)),
        out_shape=jax.ShapeDtypeStruct((Bt, S, D), jnp.float32),
    )(gathered3, pos_emb, gamma.reshape(1, D), beta.reshape(1, D))


def kernel(input_ids, word_emb, pos_emb, ln_gamma, ln_beta):
    Bt, S = input_ids.shape
    V, D = word_emb.shape
    ids = input_ids.reshape(-1).astype(jnp.int32)
    gathered = _make_sc_gather(V, D, Bt * S)(word_emb, ids)
    return _tc_add_ln(gathered.reshape(Bt, S, D), pos_emb, ln_gamma, ln_beta)


# speedup vs baseline: 1.0886x; 1.0037x over previous
import functools

import jax
import jax.numpy as jnp
from jax import lax
from jax.experimental import pallas as pl
from jax.experimental.pallas import tpu as pltpu
from jax.experimental.pallas import tpu_sc as plsc

EPS = 1e-12


def _make_sc_gather(V, D, B):
    info = plsc.get_sparse_core_info()
    NC, NS = info.num_cores, info.num_subcores
    NW = NC * NS
    assert B % NW == 0
    b_per_w = B // NW
    C = min(b_per_w, 64)
    assert b_per_w % C == 0
    n_chunks = b_per_w // C
    mesh = plsc.VectorSubcoreMesh(core_axis_name="c", subcore_axis_name="s")

    @functools.partial(
        pl.kernel,
        mesh=mesh,
        out_type=jax.ShapeDtypeStruct((B, D), jnp.float32),
        scratch_types=[
            pltpu.VMEM((C,), jnp.int32),
            pltpu.VMEM((C, D), jnp.float32),
            pltpu.SemaphoreType.DMA,
        ],
    )
    def sc_gather(table_hbm, idx_hbm, out_hbm, idx_v, rows_v, sem):
        wid = lax.axis_index("s") * NC + lax.axis_index("c")
        base = wid * b_per_w
        for c in range(n_chunks):
            lo = base + c * C
            pltpu.sync_copy(idx_hbm.at[pl.ds(lo, C)], idx_v)
            pltpu.async_copy(table_hbm.at[idx_v], rows_v, sem).wait()
            pltpu.sync_copy(rows_v, out_hbm.at[pl.ds(lo, C)])

    return sc_gather


def _tc_add_ln_body(g_ref, p_ref, gamma_ref, beta_ref, o_ref):
    x = g_ref[...] + p_ref[...][None, :, :]
    mean = jnp.mean(x, axis=-1, keepdims=True)
    xc = x - mean
    var = jnp.mean(xc * xc, axis=-1, keepdims=True)
    xhat = xc * lax.rsqrt(var + EPS)
    o_ref[...] = xhat * gamma_ref[...] + beta_ref[...]


def _tc_add_ln(gathered3, pos_emb, gamma, beta, R=2048):
    Bt, S, D = gathered3.shape
    pos_blocks = S // R
    return pl.pallas_call(
        _tc_add_ln_body,
        grid=(pos_blocks, Bt),
        in_specs=[
            pl.BlockSpec((1, R, D), lambda j, b: (b, j, 0)),
            pl.BlockSpec((R, D), lambda j, b: (j, 0)),
            pl.BlockSpec((1, D), lambda j, b: (0, 0)),
            pl.BlockSpec((1, D), lambda j, b: (0, 0)),
        ],
        out_specs=pl.BlockSpec((1, R, D), lambda j, b: (b, j, 0)),
        out_shape=jax.ShapeDtypeStruct((Bt, S, D), jnp.float32),
    )(gathered3, pos_emb, gamma.reshape(1, D), beta.reshape(1, D))


def kernel(input_ids, word_emb, pos_emb, ln_gamma, ln_beta):
    Bt, S = input_ids.shape
    V, D = word_emb.shape
    ids = input_ids.reshape(-1).astype(jnp.int32)
    gathered = _make_sc_gather(V, D, Bt * S)(word_emb, ids)
    return _tc_add_ln(gathered.reshape(Bt, S, D), pos_emb, ln_gamma, ln_beta)
